# baseline (speedup 1.0000x reference)
import functools

import jax
import jax.numpy as jnp
from jax.experimental import pallas as pl
from jax.experimental.pallas import tpu as pltpu


def _pick_bm(n: int, target: int = 400) -> int:
    best = 8
    for d in range(8, target + 1, 8):
        if n % d == 0:
            best = d
    return best


def _pass1_kernel(adj_ref, x_ref, w0_ref, b0_ref, w1_ref, b1_ref,
                  s1_ref, s0_scratch):
    @pl.when(pl.program_id(0) == 0)
    def _():
        x = jnp.maximum(x_ref[...], 0.0)
        s0_scratch[...] = (
            jnp.dot(x, w0_ref[...], preferred_element_type=jnp.float32)
            + b0_ref[...]
        )

    acc = jnp.dot(adj_ref[...], s0_scratch[...],
                  preferred_element_type=jnp.float32)
    x1 = jnp.maximum(acc, 0.0)
    s1_ref[...] = (
        jnp.dot(x1, w1_ref[...], preferred_element_type=jnp.float32)
        + b1_ref[...]
    )


def _pass2_kernel(adj_ref, s1_ref, out_ref):
    acc = jnp.dot(adj_ref[...], s1_ref[...],
                  preferred_element_type=jnp.float32)
    x2 = jnp.maximum(acc, 0.0)
    m = jnp.max(x2, axis=1, keepdims=True)
    z = x2 - m
    lse = jnp.log(jnp.sum(jnp.exp(z), axis=1, keepdims=True))
    out_ref[...] = z - lse


@jax.jit
def kernel(input, adj, W0, b0, W1, b1):
    n, in_size = input.shape
    hidd = W0.shape[1]
    n_class = W1.shape[1]
    bm = _pick_bm(n)
    grid = (n // bm,)

    b0_2d = b0.reshape(1, hidd)
    b1_2d = b1.reshape(1, n_class)

    full = lambda *shape: pl.BlockSpec(shape, lambda i: (0,) * len(shape))

    support1 = pl.pallas_call(
        _pass1_kernel,
        grid=grid,
        in_specs=[
            pl.BlockSpec((bm, n), lambda i: (i, 0)),
            full(n, in_size),
            full(in_size, hidd),
            full(1, hidd),
            full(hidd, n_class),
            full(1, n_class),
        ],
        out_specs=pl.BlockSpec((bm, n_class), lambda i: (i, 0)),
        out_shape=jax.ShapeDtypeStruct((n, n_class), jnp.float32),
        scratch_shapes=[pltpu.VMEM((n, hidd), jnp.float32)],
    )(adj, input, W0, b0_2d, W1, b1_2d)

    out = pl.pallas_call(
        _pass2_kernel,
        grid=grid,
        in_specs=[
            pl.BlockSpec((bm, n), lambda i: (i, 0)),
            full(n, n_class),
        ],
        out_specs=pl.BlockSpec((bm, n_class), lambda i: (i, 0)),
        out_shape=jax.ShapeDtypeStruct((n, n_class), jnp.float32),
    )(adj, support1)

    return out
